# baseline (speedup 1.0000x reference)
import functools

import jax
import jax.numpy as jnp
from jax import lax
from jax.experimental import pallas as pl
from jax.experimental.pallas import tpu as pltpu
from jax.experimental.pallas import tpu_sc as plsc

D_MODEL = 768
NC = 2
NS = 16
NW = NC * NS


def _make_gather(n_rows: int, d: int):
    b_per_w = n_rows // NW
    chunk = 64
    n_chunks = b_per_w // chunk

    mesh = plsc.VectorSubcoreMesh(core_axis_name="c", subcore_axis_name="s")

    @functools.partial(
        pl.kernel,
        mesh=mesh,
        out_type=jax.ShapeDtypeStruct((n_rows, d), jnp.float32),
        scratch_types=[
            pltpu.VMEM((b_per_w,), jnp.int32),
            pltpu.VMEM((chunk, d), jnp.float32),
            pltpu.VMEM((chunk, d), jnp.float32),
            pltpu.SemaphoreType.DMA,
            pltpu.SemaphoreType.DMA,
        ],
    )
    def gather_k(table_hbm, idx_hbm, out_hbm, idx_v, buf0, buf1, gsem, osem):
        wid = lax.axis_index("s") * NC + lax.axis_index("c")
        base = wid * b_per_w
        pltpu.sync_copy(idx_hbm.at[pl.ds(base, b_per_w)], idx_v)
        bufs = (buf0, buf1)
        out_copies = [None] * n_chunks
        for c in range(n_chunks):
            buf = bufs[c % 2]
            if c >= 2:
                out_copies[c - 2].wait()
            pltpu.async_copy(
                table_hbm.at[idx_v.at[pl.ds(c * chunk, chunk)]], buf, gsem
            ).wait()
            out_copies[c] = pltpu.async_copy(
                buf, out_hbm.at[pl.ds(base + c * chunk, chunk)], osem
            )
        out_copies[n_chunks - 2].wait()
        out_copies[n_chunks - 1].wait()

    return gather_k


def kernel(tokens, W_E):
    batch, seq = tokens.shape
    n_rows = batch * seq
    toks = tokens.reshape(n_rows).astype(jnp.int32)
    out = _make_gather(n_rows, D_MODEL)(W_E, toks)
    return out.reshape(batch, seq, D_MODEL)


# speedup vs baseline: 1.0529x; 1.0529x over previous
import functools

import jax
import jax.numpy as jnp
from jax import lax
from jax.experimental import pallas as pl
from jax.experimental.pallas import tpu as pltpu
from jax.experimental.pallas import tpu_sc as plsc

D_MODEL = 768
NC = 2
NS = 16
NW = NC * NS

NBUF = 4
CHUNK = 32
DEPTH = 3


def _make_gather(n_rows: int, d: int):
    b_per_w = n_rows // NW
    n_chunks = b_per_w // CHUNK

    mesh = plsc.VectorSubcoreMesh(core_axis_name="c", subcore_axis_name="s")

    @functools.partial(
        pl.kernel,
        mesh=mesh,
        out_type=jax.ShapeDtypeStruct((n_rows, d), jnp.float32),
        scratch_types=(
            [pltpu.VMEM((b_per_w,), jnp.int32)]
            + [pltpu.VMEM((CHUNK, d), jnp.float32) for _ in range(NBUF)]
            + [pltpu.SemaphoreType.DMA for _ in range(2 * NBUF)]
        ),
    )
    def gather_k(table_hbm, idx_hbm, out_hbm, idx_v, *rest):
        bufs = rest[:NBUF]
        gsems = rest[NBUF:2 * NBUF]
        osems = rest[2 * NBUF:]
        wid = lax.axis_index("s") * NC + lax.axis_index("c")
        base = wid * b_per_w
        pltpu.sync_copy(idx_hbm.at[pl.ds(base, b_per_w)], idx_v)

        gat = [None] * n_chunks
        outc = [None] * n_chunks

        def issue_gather(s):
            if s >= NBUF:
                outc[s - NBUF].wait()
            gat[s] = pltpu.async_copy(
                table_hbm.at[idx_v.at[pl.ds(s * CHUNK, CHUNK)]],
                bufs[s % NBUF], gsems[s % NBUF]
            )

        for s in range(min(DEPTH, n_chunks)):
            issue_gather(s)
        for c in range(n_chunks):
            gat[c].wait()
            outc[c] = pltpu.async_copy(
                bufs[c % NBUF],
                out_hbm.at[pl.ds(base + c * CHUNK, CHUNK)],
                osems[c % NBUF],
            )
            if c + DEPTH < n_chunks:
                issue_gather(c + DEPTH)
        for c in range(max(0, n_chunks - NBUF), n_chunks):
            outc[c].wait()

    return gather_k


def kernel(tokens, W_E):
    batch, seq = tokens.shape
    n_rows = batch * seq
    toks = tokens.reshape(n_rows).astype(jnp.int32)
    out = _make_gather(n_rows, D_MODEL)(W_E, toks)
    return out.reshape(batch, seq, D_MODEL)
